# baseline (speedup 1.0000x reference)
import functools

import jax
import jax.numpy as jnp
from jax import lax
from jax.experimental import pallas as pl
from jax.experimental.pallas import tpu as pltpu
from jax.experimental.pallas import tpu_sc as plsc

N_P = 10000
E = 320000
D = 128
NC = 2
NS = 16
NW = NC * NS
PER_TILE = E // NW
WIN = 80
NWIN = PER_TILE // WIN
N_PAD = 10240
ROWS_PER_SUB = N_PAD // NS

_sc_mesh = plsc.VectorSubcoreMesh(core_axis_name="c", subcore_axis_name="s")


@functools.partial(
    pl.kernel,
    out_type=[jax.ShapeDtypeStruct((E, D), jnp.float32),
              jax.ShapeDtypeStruct((E, D), jnp.float32)],
    mesh=_sc_mesh,
    scratch_types=[
        pltpu.VMEM((WIN,), jnp.int32),
        pltpu.VMEM((WIN,), jnp.int32),
        pltpu.VMEM((WIN, D), jnp.float32),
        pltpu.VMEM((WIN, D), jnp.float32),
        pltpu.SemaphoreType.DMA,
        pltpu.SemaphoreType.DMA,
    ],
)
def _sc_gather(xp_hbm, xc_hbm, dst_hbm, src_hbm, gp_hbm, gc_hbm,
               idxd_v, idxs_v, rowp_v, rowc_v, sem1, sem2):
    wid = lax.axis_index("s") * NC + lax.axis_index("c")
    base0 = wid * PER_TILE

    @pl.loop(0, NWIN)
    def _(w):
        base = base0 + w * WIN
        pltpu.sync_copy(dst_hbm.at[pl.ds(base, WIN)], idxd_v)
        pltpu.sync_copy(src_hbm.at[pl.ds(base, WIN)], idxs_v)
        cp1 = pltpu.async_copy(xp_hbm.at[idxd_v], rowp_v, sem1)
        cp2 = pltpu.async_copy(xc_hbm.at[idxs_v], rowc_v, sem2)
        cp1.wait()
        cp2.wait()
        pltpu.sync_copy(rowp_v, gp_hbm.at[pl.ds(base, WIN)])
        pltpu.sync_copy(rowc_v, gc_hbm.at[pl.ds(base, WIN)])


EBLK = 3200


def _tc_body(gp_ref, gc_ref, amf_ref, ea_ref, w1p_ref, w1c_ref, w1a_ref,
             w2_ref, out_ref):
    t1 = jnp.dot(gp_ref[...].astype(jnp.bfloat16), w1p_ref[...],
                 preferred_element_type=jnp.float32)
    t1 += jnp.dot(gc_ref[...].astype(jnp.bfloat16), w1c_ref[...],
                  preferred_element_type=jnp.float32)
    t1 += jnp.dot(amf_ref[...].astype(jnp.bfloat16), w1a_ref[...],
                  preferred_element_type=jnp.float32)
    ea = ea_ref[...]
    pre1 = (ea[:, 0:1] * t1[:, 0:128] + ea[:, 1:2] * t1[:, 128:256]
            + ea[:, 2:3] * t1[:, 256:384] + ea[:, 3:4] * t1[:, 384:512])
    m1 = pre1 * jax.nn.sigmoid(pre1)
    t2 = jnp.dot(m1.astype(jnp.bfloat16), w2_ref[...],
                 preferred_element_type=jnp.float32)
    pre2 = (ea[:, 0:1] * t2[:, 0:128] + ea[:, 1:2] * t2[:, 128:256]
            + ea[:, 2:3] * t2[:, 256:384] + ea[:, 3:4] * t2[:, 384:512])
    out_ref[...] = pre2 * jax.nn.sigmoid(pre2)


def _tc_messages(gp, gc, amf, ea, w1p, w1c, w1a, w2r):
    grid = (E // EBLK,)
    return pl.pallas_call(
        _tc_body,
        grid=grid,
        in_specs=[
            pl.BlockSpec((EBLK, D), lambda i: (i, 0)),
            pl.BlockSpec((EBLK, D), lambda i: (i, 0)),
            pl.BlockSpec((EBLK, 16), lambda i: (i, 0)),
            pl.BlockSpec((EBLK, 4), lambda i: (i, 0)),
            pl.BlockSpec((D, 512), lambda i: (0, 0)),
            pl.BlockSpec((D, 512), lambda i: (0, 0)),
            pl.BlockSpec((16, 512), lambda i: (0, 0)),
            pl.BlockSpec((D, 512), lambda i: (0, 0)),
        ],
        out_specs=pl.BlockSpec((EBLK, D), lambda i: (i, 0)),
        out_shape=jax.ShapeDtypeStruct((E, D), jnp.float32),
    )(gp, gc, amf, ea, w1p, w1c, w1a, w2r)


@functools.partial(
    pl.kernel,
    out_type=jax.ShapeDtypeStruct((NC, N_PAD, D), jnp.float32),
    mesh=_sc_mesh,
    scratch_types=[
        pltpu.VMEM((WIN,), jnp.int32),
        pltpu.VMEM((WIN, D), jnp.float32),
        pltpu.VMEM_SHARED((N_PAD, D), jnp.float32),
    ],
)
def _sc_scatter(m2_hbm, dst_hbm, zeros_hbm, out_hbm, idx_v, rows_v, acc_sh):
    cid = lax.axis_index("c")
    sid = lax.axis_index("s")
    wid = sid * NC + cid
    r0 = sid * ROWS_PER_SUB
    pltpu.sync_copy(zeros_hbm.at[pl.ds(r0, ROWS_PER_SUB)],
                    acc_sh.at[pl.ds(r0, ROWS_PER_SUB)])
    plsc.subcore_barrier()

    base0 = wid * PER_TILE

    @pl.loop(0, NWIN)
    def _(w):
        base = base0 + w * WIN
        pltpu.sync_copy(dst_hbm.at[pl.ds(base, WIN)], idx_v)
        pltpu.sync_copy(m2_hbm.at[pl.ds(base, WIN)], rows_v)
        pltpu.sync_copy(rows_v, acc_sh.at[idx_v], add=True)

    plsc.subcore_barrier()
    pltpu.sync_copy(acc_sh.at[pl.ds(r0, ROWS_PER_SUB)],
                    out_hbm.at[cid].at[pl.ds(r0, ROWS_PER_SUB)])


RBLK = 2000


def _combine_body(xp_ref, parts_ref, out_ref):
    out_ref[:, 0:D] = xp_ref[...]
    out_ref[:, D:2 * D] = parts_ref[0] + parts_ref[1]


def _tc_combine(x_p, parts):
    return pl.pallas_call(
        _combine_body,
        grid=(N_P // RBLK,),
        in_specs=[
            pl.BlockSpec((RBLK, D), lambda i: (i, 0)),
            pl.BlockSpec((NC, RBLK, D), lambda i: (0, i, 0)),
        ],
        out_specs=pl.BlockSpec((RBLK, 2 * D), lambda i: (i, 0)),
        out_shape=jax.ShapeDtypeStruct((N_P, 2 * D), jnp.float32),
    )(x_p, parts)


def kernel(x_p, x_c, edge_index, edge_attr, batch, additional_message_features,
           W1, W2):
    del batch
    src = edge_index[0].astype(jnp.int32)
    dst = edge_index[1].astype(jnp.int32)
    w1r = W1.reshape(272, 512).astype(jnp.bfloat16)
    w1p = w1r[0:128]
    w1c = w1r[128:256]
    w1a = w1r[256:272]
    w2r = W2.reshape(128, 512).astype(jnp.bfloat16)

    gp, gc = _sc_gather(x_p, x_c, dst, src)
    m2 = _tc_messages(gp, gc, additional_message_features, edge_attr,
                      w1p, w1c, w1a, w2r)
    zeros = jnp.zeros((N_PAD, D), jnp.float32)
    parts = _sc_scatter(m2, dst, zeros)
    return _tc_combine(x_p, parts)


# speedup vs baseline: 1.3004x; 1.3004x over previous
import functools

import jax
import jax.numpy as jnp
from jax import lax
from jax.experimental import pallas as pl
from jax.experimental.pallas import tpu as pltpu
from jax.experimental.pallas import tpu_sc as plsc

N_P = 10000
E = 320000
D = 128
NC = 2
NS = 16
NW = NC * NS
WIN = 128
N_PAD = 10240
ROWS_PER_SUB = N_PAD // NS

_sc_mesh = plsc.VectorSubcoreMesh(core_axis_name="c", subcore_axis_name="s")


NWTOT = E // WIN


@functools.partial(
    pl.kernel,
    out_type=[jax.ShapeDtypeStruct((E, D), jnp.float32),
              jax.ShapeDtypeStruct((E, D), jnp.float32)],
    mesh=_sc_mesh,
)
def _sc_gather(xp_hbm, xc_hbm, dst_hbm, src_hbm, gp_hbm, gc_hbm):
    def body(dstb, srcb, gpb, gcb):
        pltpu.sync_copy(xp_hbm.at[dstb.at[0]], gpb)
        pltpu.sync_copy(xc_hbm.at[srcb.at[0]], gcb)

    pltpu.emit_pipeline(
        body,
        grid=(NWTOT,),
        in_specs=[
            pl.BlockSpec((1, WIN), lambda i: (0, i)),
            pl.BlockSpec((1, WIN), lambda i: (0, i)),
        ],
        out_specs=[
            pl.BlockSpec((WIN, D), lambda i: (i, 0)),
            pl.BlockSpec((WIN, D), lambda i: (i, 0)),
        ],
        core_axis_name=("c", "s"),
        dimension_semantics=(pltpu.PARALLEL,),
    )(dst_hbm, src_hbm, gp_hbm, gc_hbm)


EBLK = 3200


def _tc_body(gp_ref, gc_ref, amf_ref, ea_ref, w1p_ref, w1c_ref, w1a_ref,
             w2_ref, out_ref):
    t1 = jnp.dot(gp_ref[...].astype(jnp.bfloat16), w1p_ref[...],
                 preferred_element_type=jnp.float32)
    t1 += jnp.dot(gc_ref[...].astype(jnp.bfloat16), w1c_ref[...],
                  preferred_element_type=jnp.float32)
    t1 += jnp.dot(amf_ref[...].astype(jnp.bfloat16), w1a_ref[...],
                  preferred_element_type=jnp.float32)
    ea = ea_ref[...]
    pre1 = (ea[:, 0:1] * t1[:, 0:128] + ea[:, 1:2] * t1[:, 128:256]
            + ea[:, 2:3] * t1[:, 256:384] + ea[:, 3:4] * t1[:, 384:512])
    m1 = pre1 * jax.nn.sigmoid(pre1)
    t2 = jnp.dot(m1.astype(jnp.bfloat16), w2_ref[...],
                 preferred_element_type=jnp.float32)
    pre2 = (ea[:, 0:1] * t2[:, 0:128] + ea[:, 1:2] * t2[:, 128:256]
            + ea[:, 2:3] * t2[:, 256:384] + ea[:, 3:4] * t2[:, 384:512])
    out_ref[...] = pre2 * jax.nn.sigmoid(pre2)


def _tc_messages(gp, gc, amf, ea, w1p, w1c, w1a, w2r):
    grid = (E // EBLK,)
    return pl.pallas_call(
        _tc_body,
        grid=grid,
        in_specs=[
            pl.BlockSpec((EBLK, D), lambda i: (i, 0)),
            pl.BlockSpec((EBLK, D), lambda i: (i, 0)),
            pl.BlockSpec((EBLK, 16), lambda i: (i, 0)),
            pl.BlockSpec((EBLK, 4), lambda i: (i, 0)),
            pl.BlockSpec((D, 512), lambda i: (0, 0)),
            pl.BlockSpec((D, 512), lambda i: (0, 0)),
            pl.BlockSpec((16, 512), lambda i: (0, 0)),
            pl.BlockSpec((D, 512), lambda i: (0, 0)),
        ],
        out_specs=pl.BlockSpec((EBLK, D), lambda i: (i, 0)),
        out_shape=jax.ShapeDtypeStruct((E, D), jnp.float32),
    )(gp, gc, amf, ea, w1p, w1c, w1a, w2r)


@functools.partial(
    pl.kernel,
    out_type=jax.ShapeDtypeStruct((NC, N_PAD, D), jnp.float32),
    mesh=_sc_mesh,
    scratch_types=[
        pltpu.VMEM_SHARED((N_PAD, D), jnp.float32),
    ],
)
def _sc_scatter(m2_hbm, dst_hbm, zeros_hbm, out_hbm, acc_sh):
    cid = lax.axis_index("c")
    sid = lax.axis_index("s")
    r0 = sid * ROWS_PER_SUB
    pltpu.sync_copy(zeros_hbm.at[pl.ds(r0, ROWS_PER_SUB)],
                    acc_sh.at[pl.ds(r0, ROWS_PER_SUB)])
    plsc.subcore_barrier()

    def body(m2b, idxb):
        pltpu.sync_copy(m2b, acc_sh.at[idxb.at[0]], add=True)

    pltpu.emit_pipeline(
        body,
        grid=(NWTOT,),
        in_specs=[
            pl.BlockSpec((WIN, D), lambda i: (i, 0)),
            pl.BlockSpec((1, WIN), lambda i: (0, i)),
        ],
        out_specs=[],
        core_axis_name=("c", "s"),
        dimension_semantics=(pltpu.PARALLEL,),
    )(m2_hbm, dst_hbm)

    plsc.subcore_barrier()
    pltpu.sync_copy(acc_sh.at[pl.ds(r0, ROWS_PER_SUB)],
                    out_hbm.at[cid].at[pl.ds(r0, ROWS_PER_SUB)])


RBLK = 2000


def _combine_body(xp_ref, parts_ref, out_ref):
    out_ref[:, 0:D] = xp_ref[...]
    out_ref[:, D:2 * D] = parts_ref[0] + parts_ref[1]


def _tc_combine(x_p, parts):
    return pl.pallas_call(
        _combine_body,
        grid=(N_P // RBLK,),
        in_specs=[
            pl.BlockSpec((RBLK, D), lambda i: (i, 0)),
            pl.BlockSpec((NC, RBLK, D), lambda i: (0, i, 0)),
        ],
        out_specs=pl.BlockSpec((RBLK, 2 * D), lambda i: (i, 0)),
        out_shape=jax.ShapeDtypeStruct((N_P, 2 * D), jnp.float32),
    )(x_p, parts)


def kernel(x_p, x_c, edge_index, edge_attr, batch, additional_message_features,
           W1, W2):
    del batch
    src = edge_index[0].astype(jnp.int32)
    dst = edge_index[1].astype(jnp.int32)
    w1r = W1.reshape(272, 512).astype(jnp.bfloat16)
    w1p = w1r[0:128]
    w1c = w1r[128:256]
    w1a = w1r[256:272]
    w2r = W2.reshape(128, 512).astype(jnp.bfloat16)

    dst2 = dst.reshape(1, E)
    src2 = src.reshape(1, E)
    gp, gc = _sc_gather(x_p, x_c, dst2, src2)
    m2 = _tc_messages(gp, gc, additional_message_features, edge_attr,
                      w1p, w1c, w1a, w2r)
    zeros = jnp.zeros((N_PAD, D), jnp.float32)
    parts = _sc_scatter(m2, dst2, zeros)
    return _tc_combine(x_p, parts)


# speedup vs baseline: 1.3530x; 1.0405x over previous
import functools

import jax
import jax.numpy as jnp
from jax import lax
from jax.experimental import pallas as pl
from jax.experimental.pallas import tpu as pltpu
from jax.experimental.pallas import tpu_sc as plsc

N_P = 10000
E = 320000
D = 128
NC = 2
NS = 16
NW = NC * NS
WIN = 128
N_PAD = 10240
ROWS_PER_SUB = N_PAD // NS

_sc_mesh = plsc.VectorSubcoreMesh(core_axis_name="c", subcore_axis_name="s")


NWTOT = E // WIN


@functools.partial(
    pl.kernel,
    out_type=[jax.ShapeDtypeStruct((E, D), jnp.float32),
              jax.ShapeDtypeStruct((E, D), jnp.float32)],
    mesh=_sc_mesh,
)
def _sc_gather(xp_hbm, xc_hbm, dst_hbm, src_hbm, gp_hbm, gc_hbm):
    def body(dstb, srcb, gpb, gcb):
        pltpu.sync_copy(xp_hbm.at[dstb.at[0]], gpb)
        pltpu.sync_copy(xc_hbm.at[srcb.at[0]], gcb)

    pltpu.emit_pipeline(
        body,
        grid=(NWTOT,),
        in_specs=[
            pl.BlockSpec((1, WIN), lambda i: (0, i)),
            pl.BlockSpec((1, WIN), lambda i: (0, i)),
        ],
        out_specs=[
            pl.BlockSpec((WIN, D), lambda i: (i, 0)),
            pl.BlockSpec((WIN, D), lambda i: (i, 0)),
        ],
        core_axis_name=("c", "s"),
        dimension_semantics=(pltpu.PARALLEL,),
    )(dst_hbm, src_hbm, gp_hbm, gc_hbm)


EBLK = 3200


def _tc_body(gp_ref, gc_ref, amf_ref, ea_ref, w1_ref, w2_ref, out_ref):
    feat = jnp.concatenate(
        [gp_ref[...].astype(jnp.bfloat16),
         gc_ref[...].astype(jnp.bfloat16),
         amf_ref[...].astype(jnp.bfloat16)], axis=1)
    t1 = jnp.dot(feat, w1_ref[...], preferred_element_type=jnp.float32)
    ea = ea_ref[...]
    eab = [jnp.broadcast_to(ea[:, j:j + 1], (EBLK, D)) for j in range(4)]
    pre1 = (eab[0] * t1[:, 0:128] + eab[1] * t1[:, 128:256]
            + eab[2] * t1[:, 256:384] + eab[3] * t1[:, 384:512])
    m1 = pre1 * jax.nn.sigmoid(pre1)
    t2 = jnp.dot(m1.astype(jnp.bfloat16), w2_ref[...],
                 preferred_element_type=jnp.float32)
    pre2 = (eab[0] * t2[:, 0:128] + eab[1] * t2[:, 128:256]
            + eab[2] * t2[:, 256:384] + eab[3] * t2[:, 384:512])
    out_ref[...] = pre2 * jax.nn.sigmoid(pre2)


def _tc_messages(gp, gc, amf, ea, w1r, w2r):
    grid = (E // EBLK,)
    return pl.pallas_call(
        _tc_body,
        grid=grid,
        in_specs=[
            pl.BlockSpec((EBLK, D), lambda i: (i, 0)),
            pl.BlockSpec((EBLK, D), lambda i: (i, 0)),
            pl.BlockSpec((EBLK, 16), lambda i: (i, 0)),
            pl.BlockSpec((EBLK, 4), lambda i: (i, 0)),
            pl.BlockSpec((272, 512), lambda i: (0, 0)),
            pl.BlockSpec((D, 512), lambda i: (0, 0)),
        ],
        out_specs=pl.BlockSpec((EBLK, D), lambda i: (i, 0)),
        out_shape=jax.ShapeDtypeStruct((E, D), jnp.float32),
    )(gp, gc, amf, ea, w1r, w2r)


@functools.partial(
    pl.kernel,
    out_type=jax.ShapeDtypeStruct((NC, N_PAD, D), jnp.float32),
    mesh=_sc_mesh,
    scratch_types=[
        pltpu.VMEM_SHARED((N_PAD, D), jnp.float32),
    ],
)
def _sc_scatter(m2_hbm, dst_hbm, zeros_hbm, out_hbm, acc_sh):
    cid = lax.axis_index("c")
    sid = lax.axis_index("s")
    r0 = sid * ROWS_PER_SUB
    pltpu.sync_copy(zeros_hbm.at[pl.ds(r0, ROWS_PER_SUB)],
                    acc_sh.at[pl.ds(r0, ROWS_PER_SUB)])
    plsc.subcore_barrier()

    def body(m2b, idxb):
        pltpu.sync_copy(m2b, acc_sh.at[idxb.at[0]], add=True)

    pltpu.emit_pipeline(
        body,
        grid=(NWTOT,),
        in_specs=[
            pl.BlockSpec((WIN, D), lambda i: (i, 0)),
            pl.BlockSpec((1, WIN), lambda i: (0, i)),
        ],
        out_specs=[],
        core_axis_name=("c", "s"),
        dimension_semantics=(pltpu.PARALLEL,),
    )(m2_hbm, dst_hbm)

    plsc.subcore_barrier()
    pltpu.sync_copy(acc_sh.at[pl.ds(r0, ROWS_PER_SUB)],
                    out_hbm.at[cid].at[pl.ds(r0, ROWS_PER_SUB)])


RBLK = 2000


def _combine_body(xp_ref, parts_ref, out_ref):
    out_ref[:, 0:D] = xp_ref[...]
    out_ref[:, D:2 * D] = parts_ref[0] + parts_ref[1]


def _tc_combine(x_p, parts):
    return pl.pallas_call(
        _combine_body,
        grid=(N_P // RBLK,),
        in_specs=[
            pl.BlockSpec((RBLK, D), lambda i: (i, 0)),
            pl.BlockSpec((NC, RBLK, D), lambda i: (0, i, 0)),
        ],
        out_specs=pl.BlockSpec((RBLK, 2 * D), lambda i: (i, 0)),
        out_shape=jax.ShapeDtypeStruct((N_P, 2 * D), jnp.float32),
    )(x_p, parts)


def kernel(x_p, x_c, edge_index, edge_attr, batch, additional_message_features,
           W1, W2):
    del batch
    src = edge_index[0].astype(jnp.int32)
    dst = edge_index[1].astype(jnp.int32)
    w1r = W1.reshape(272, 512).astype(jnp.bfloat16)
    w2r = W2.reshape(128, 512).astype(jnp.bfloat16)

    dst2 = dst.reshape(1, E)
    src2 = src.reshape(1, E)
    gp, gc = _sc_gather(x_p, x_c, dst2, src2)
    m2 = _tc_messages(gp, gc, additional_message_features, edge_attr,
                      w1r, w2r)
    zeros = jnp.zeros((N_PAD, D), jnp.float32)
    parts = _sc_scatter(m2, dst2, zeros)
    return _tc_combine(x_p, parts)
